# baseline (speedup 1.0000x reference)
import functools

import jax
import jax.numpy as jnp
from jax.experimental import pallas as pl

EPSILON = 1.0
EPS = 1e-8

N = 10000
H = 128


def _stage1_body(s_ref, Wi1_ref, bi1_ref, Wi2_ref, bi2_ref, x_ref):
    s = s_ref[...]
    h = jnp.dot(s, Wi1_ref[...], preferred_element_type=jnp.float32) + bi1_ref[...]
    h = h * jax.nn.sigmoid(h)
    x_ref[...] = jnp.dot(h, Wi2_ref[...], preferred_element_type=jnp.float32) + bi2_ref[...]


def _stage1(s2d, Wi1, bi1, Wi2, bi2, block=1000):
    n = s2d.shape[0]
    grid = (n // block,)
    return pl.pallas_call(
        _stage1_body,
        grid=grid,
        in_specs=[
            pl.BlockSpec((block, H), lambda i: (i, 0)),
            pl.BlockSpec((H, H), lambda i: (0, 0)),
            pl.BlockSpec((H,), lambda i: (0,)),
            pl.BlockSpec((H, 3 * H), lambda i: (0, 0)),
            pl.BlockSpec((3 * H,), lambda i: (0,)),
        ],
        out_specs=pl.BlockSpec((block, 3 * H), lambda i: (i, 0)),
        out_shape=jax.ShapeDtypeStruct((n, 3 * H), jnp.float32),
    )(s2d, Wi1, bi1, Wi2, bi2)


def _stage3_body(s_ref, v_ref, Wm1_ref, bm1_ref, Wm2_ref, bm2_ref, Wvm_ref,
                 so_ref, vo_ref, *, block):
    s = s_ref[...]
    v = v_ref[...].reshape(block, 3, H)
    vm = jnp.dot(v.reshape(block * 3, H), Wvm_ref[...],
                 preferred_element_type=jnp.float32).reshape(block, 3, 2 * H)
    v_l = vm[:, :, :H]
    v_r = vm[:, :, H:]
    v_norm = jnp.sqrt(jnp.sum(v_r * v_r, axis=1) + EPS)
    ts = jnp.concatenate([s, v_norm], axis=-1)
    hmid = jnp.dot(ts, Wm1_ref[...], preferred_element_type=jnp.float32) + bm1_ref[...]
    hmid = hmid * jax.nn.sigmoid(hmid)
    hh = jnp.dot(hmid, Wm2_ref[...], preferred_element_type=jnp.float32) + bm2_ref[...]
    ds2 = hh[:, :H]
    dv2u = hh[:, H:2 * H]
    dsv = hh[:, 2 * H:]
    dotrl = jnp.sum(v_r * v_l, axis=1)
    so_ref[...] = (s + ds2 + dsv * dotrl) * EPSILON
    vo_ref[...] = ((v + v_l * dv2u[:, None, :]) * EPSILON).reshape(block, 3 * H)


def _stage3(s2d, v2d, Wm1, bm1, Wm2, bm2, Wvm, block=1000):
    n = s2d.shape[0]
    grid = (n // block,)
    return pl.pallas_call(
        functools.partial(_stage3_body, block=block),
        grid=grid,
        in_specs=[
            pl.BlockSpec((block, H), lambda i: (i, 0)),
            pl.BlockSpec((block, 3 * H), lambda i: (i, 0)),
            pl.BlockSpec((2 * H, H), lambda i: (0, 0)),
            pl.BlockSpec((H,), lambda i: (0,)),
            pl.BlockSpec((H, 3 * H), lambda i: (0, 0)),
            pl.BlockSpec((3 * H,), lambda i: (0,)),
            pl.BlockSpec((H, 2 * H), lambda i: (0, 0)),
        ],
        out_specs=[
            pl.BlockSpec((block, H), lambda i: (i, 0)),
            pl.BlockSpec((block, 3 * H), lambda i: (i, 0)),
        ],
        out_shape=[
            jax.ShapeDtypeStruct((n, H), jnp.float32),
            jax.ShapeDtypeStruct((n, 3 * H), jnp.float32),
        ],
    )(s2d, v2d, Wm1, bm1, Wm2, bm2, Wvm)


def kernel(s, v, dir_ij, Wij, senders, receivers,
           Wi1, bi1, Wi2, bi2, Wm1, bm1, Wm2, bm2, Wvm):
    n_nodes = s.shape[0]
    s2d = s.reshape(n_nodes, H)
    x = _stage1(s2d, Wi1, bi1, Wi2, bi2)

    xj = x[receivers]
    vj = v[receivers]
    m = Wij.reshape(-1, 3 * H) * xj
    ds_e = m[:, :H]
    dv1 = m[:, H:2 * H]
    dv2 = m[:, 2 * H:]
    dv_e = dv1[:, None, :] * dir_ij[:, :, None] + dv2[:, None, :] * vj
    ds = jax.ops.segment_sum(ds_e, senders, num_segments=n_nodes)
    dv = jax.ops.segment_sum(dv_e, senders, num_segments=n_nodes)

    s2 = (s2d + ds) * EPSILON
    v2 = (v.reshape(n_nodes, 3 * H) + dv.reshape(n_nodes, 3 * H)) * EPSILON

    so, vo = _stage3(s2, v2, Wm1, bm1, Wm2, bm2, Wvm)
    return (so.reshape(n_nodes, 1, H), vo.reshape(n_nodes, 3, H))


# speedup vs baseline: 2.7885x; 2.7885x over previous
import functools

import jax
import jax.numpy as jnp
from jax import lax
from jax.experimental import pallas as pl
from jax.experimental.pallas import tpu as pltpu
from jax.experimental.pallas import tpu_sc as plsc

EPSILON = 1.0
EPS = 1e-8

N = 10000
E = 160000
H = 128

NC = 2
NS = 16
NW = NC * NS
NR = 2
NACC = 160
NPW = NR * NACC
NPADR = NW * NPW
SCH = 1024
NSCH = (E + SCH - 1) // SCH
EPAD = (NSCH + 1) * SCH
CB = 32
TRASH = SCH + CB
BKTW = SCH + CB + 16
F = 3 * H
OF = 4 * H


def _i16(val):
    return jnp.broadcast_to(val, (16,)).astype(jnp.int32)


def _sc_edge_body(x_hbm, v_hbm, wij_hbm, pk_hbm, snd_hbm, out_hbm,
                  sbuf, bkt, ibuf, rbuf, pkbuf, wbuf, xbuf, vbuf, acc,
                  sem_s, sem_pk, sem_w, sem_g):
    core = lax.axis_index("c")
    sub = lax.axis_index("s")
    w = core * NS + sub
    lane = lax.iota(jnp.int32, 16)
    z16 = jnp.zeros((16,), jnp.float32)

    for r in range(NR):
        lo = w * NPW + r * NACC

        def zrow(i, _):
            for g in range(OF // 16):
                acc[i, pl.ds(g * 16, 16)] = z16
            return 0

        lax.fori_loop(0, NACC + 1, zrow, 0)

        pltpu.sync_copy(snd_hbm.at[pl.ds(0, SCH)], sbuf.at[0])

        def scanchunk(sc, _, lo=lo):
            cp_n = pltpu.async_copy(
                snd_hbm.at[pl.ds((sc + 1) * SCH, SCH)],
                sbuf.at[(sc + 1) % 2], sem_s)

            def groupfn(g, cnt, sc=sc, lo=lo):
                sv = sbuf[sc % 2, pl.ds(g * 16, 16)]
                eid = _i16(sc * SCH + g * 16) + lane
                m = jnp.logical_and(sv >= _i16(lo), sv < _i16(lo + NACC))
                mi = m.astype(jnp.int32)
                offs = plsc.cumsum(mi)
                tgt = jnp.where(m, _i16(cnt) + offs - 1, _i16(TRASH))
                plsc.store_scatter(bkt, [tgt], eid)
                return cnt + jnp.sum(mi)

            cnt = lax.fori_loop(0, SCH // 16, groupfn, 0)

            end = ((cnt + CB - 1) // CB) * CB
            for rr in range(CB // 16):
                ii = _i16(cnt + rr * 16) + lane
                tgt = jnp.where(ii < _i16(end), ii, _i16(TRASH))
                plsc.store_scatter(bkt, [tgt], jnp.zeros((16,), jnp.int32))

            nch = (cnt + CB - 1) // CB

            def chunkfn(ci, _, cnt=cnt, lo=lo):
                base = ci * CB
                for s2 in range(CB // 16):
                    ibuf[pl.ds(s2 * 16, 16)] = bkt[pl.ds(base + s2 * 16, 16)]
                cp_pk = pltpu.async_copy(pk_hbm.at[ibuf], pkbuf, sem_pk)
                cp_w = pltpu.async_copy(wij_hbm.at[ibuf], wbuf, sem_w)
                cp_pk.wait()
                for s2 in range(CB // 16):
                    rl = lane + s2 * 16
                    rv = plsc.load_gather(pkbuf, [rl, _i16(0)]).astype(jnp.int32)
                    rbuf[pl.ds(s2 * 16, 16)] = rv
                cp_x = pltpu.async_copy(x_hbm.at[rbuf], xbuf, sem_g)
                cp_v = pltpu.async_copy(v_hbm.at[rbuf], vbuf, sem_g)
                cp_w.wait()
                cp_x.wait()
                cp_v.wait()

                def edgefn(e, _, cnt=cnt, lo=lo, base=base):
                    validv = _i16(base + e) < _i16(cnt)
                    sndl = plsc.load_gather(
                        pkbuf, [_i16(e), _i16(4)]).astype(jnp.int32)
                    rowv = jnp.where(validv, sndl - _i16(lo), _i16(NACC))
                    d0 = plsc.load_gather(pkbuf, [_i16(e), _i16(1)])
                    d1 = plsc.load_gather(pkbuf, [_i16(e), _i16(2)])
                    d2 = plsc.load_gather(pkbuf, [_i16(e), _i16(3)])
                    m = []
                    for g in range(F // 16):
                        wv = wbuf[e, pl.ds(g * 16, 16)]
                        xv = xbuf[e, pl.ds(g * 16, 16)]
                        m.append(wv * xv)
                    for g in range(8):
                        plsc.addupdate_scatter(
                            acc, [rowv, _i16(g * 16) + lane], m[g])
                    for k, dk in enumerate((d0, d1, d2)):
                        for g in range(8):
                            vv = vbuf[e, pl.ds(k * 128 + g * 16, 16)]
                            val = m[8 + g] * dk + m[16 + g] * vv
                            plsc.addupdate_scatter(
                                acc, [rowv, _i16(128 + k * 128 + g * 16) + lane],
                                val)
                    return 0

                lax.fori_loop(0, CB, edgefn, 0)
                return 0

            lax.fori_loop(0, nch, chunkfn, 0)
            cp_n.wait()
            return 0

        lax.fori_loop(0, NSCH, scanchunk, 0)

        pltpu.sync_copy(acc.at[pl.ds(0, NACC)],
                        out_hbm.at[pl.ds(lo, NACC)])


def _sc_edge(x, v2d, wij2d, pk, spad):
    mesh = plsc.VectorSubcoreMesh(core_axis_name="c", subcore_axis_name="s")
    fn = pl.kernel(
        _sc_edge_body,
        out_type=jax.ShapeDtypeStruct((NPADR, OF), jnp.float32),
        mesh=mesh,
        compiler_params=pltpu.CompilerParams(needs_layout_passes=False),
        scratch_types=[
            pltpu.VMEM((2, SCH), jnp.int32),
            pltpu.VMEM((BKTW,), jnp.int32),
            pltpu.VMEM((CB,), jnp.int32),
            pltpu.VMEM((CB,), jnp.int32),
            pltpu.VMEM((CB, 128), jnp.float32),
            pltpu.VMEM((CB, F), jnp.float32),
            pltpu.VMEM((CB, F), jnp.float32),
            pltpu.VMEM((CB, F), jnp.float32),
            pltpu.VMEM((NACC + 1, OF), jnp.float32),
            pltpu.SemaphoreType.DMA,
            pltpu.SemaphoreType.DMA,
            pltpu.SemaphoreType.DMA,
            pltpu.SemaphoreType.DMA,
        ],
    )
    return fn(x, v2d, wij2d, pk, spad)


def _stage1_body(s_ref, Wi1_ref, bi1_ref, Wi2_ref, bi2_ref, x_ref):
    s = s_ref[...]
    h = jnp.dot(s, Wi1_ref[...], preferred_element_type=jnp.float32) + bi1_ref[...]
    h = h * jax.nn.sigmoid(h)
    x_ref[...] = jnp.dot(h, Wi2_ref[...], preferred_element_type=jnp.float32) + bi2_ref[...]


def _stage1(s2d, Wi1, bi1, Wi2, bi2, block=1000):
    n = s2d.shape[0]
    grid = (n // block,)
    return pl.pallas_call(
        _stage1_body,
        grid=grid,
        in_specs=[
            pl.BlockSpec((block, H), lambda i: (i, 0)),
            pl.BlockSpec((H, H), lambda i: (0, 0)),
            pl.BlockSpec((H,), lambda i: (0,)),
            pl.BlockSpec((H, 3 * H), lambda i: (0, 0)),
            pl.BlockSpec((3 * H,), lambda i: (0,)),
        ],
        out_specs=pl.BlockSpec((block, 3 * H), lambda i: (i, 0)),
        out_shape=jax.ShapeDtypeStruct((n, 3 * H), jnp.float32),
    )(s2d, Wi1, bi1, Wi2, bi2)


def _stage3_body(s_ref, v_ref, o_ref, Wm1_ref, bm1_ref, Wm2_ref,
                 bm2_ref, Wvm_ref, so_ref, vo_ref, *, block):
    s0 = s_ref[...]
    v0 = v_ref[...].reshape(block, 3, H)
    o = o_ref[...]
    s = (s0 + o[:, :H]) * EPSILON
    v = (v0 + o[:, H:].reshape(block, 3, H)) * EPSILON
    vm = jnp.dot(v.reshape(block * 3, H), Wvm_ref[...],
                 preferred_element_type=jnp.float32).reshape(block, 3, 2 * H)
    v_l = vm[:, :, :H]
    v_r = vm[:, :, H:]
    v_norm = jnp.sqrt(jnp.sum(v_r * v_r, axis=1) + EPS)
    ts = jnp.concatenate([s, v_norm], axis=-1)
    hmid = jnp.dot(ts, Wm1_ref[...], preferred_element_type=jnp.float32) + bm1_ref[...]
    hmid = hmid * jax.nn.sigmoid(hmid)
    hh = jnp.dot(hmid, Wm2_ref[...], preferred_element_type=jnp.float32) + bm2_ref[...]
    ds2 = hh[:, :H]
    dv2u = hh[:, H:2 * H]
    dsv = hh[:, 2 * H:]
    dotrl = jnp.sum(v_r * v_l, axis=1)
    so_ref[...] = (s + ds2 + dsv * dotrl) * EPSILON
    vo_ref[...] = ((v + v_l * dv2u[:, None, :]) * EPSILON).reshape(block, 3 * H)


def _stage3(s2d, v2d, out, Wm1, bm1, Wm2, bm2, Wvm, block=1000):
    n = s2d.shape[0]
    grid = (n // block,)
    return pl.pallas_call(
        functools.partial(_stage3_body, block=block),
        grid=grid,
        in_specs=[
            pl.BlockSpec((block, H), lambda i: (i, 0)),
            pl.BlockSpec((block, 3 * H), lambda i: (i, 0)),
            pl.BlockSpec((block, OF), lambda i: (i, 0)),
            pl.BlockSpec((2 * H, H), lambda i: (0, 0)),
            pl.BlockSpec((H,), lambda i: (0,)),
            pl.BlockSpec((H, 3 * H), lambda i: (0, 0)),
            pl.BlockSpec((3 * H,), lambda i: (0,)),
            pl.BlockSpec((H, 2 * H), lambda i: (0, 0)),
        ],
        out_specs=[
            pl.BlockSpec((block, H), lambda i: (i, 0)),
            pl.BlockSpec((block, 3 * H), lambda i: (i, 0)),
        ],
        out_shape=[
            jax.ShapeDtypeStruct((n, H), jnp.float32),
            jax.ShapeDtypeStruct((n, 3 * H), jnp.float32),
        ],
    )(s2d, v2d, out, Wm1, bm1, Wm2, bm2, Wvm)


def kernel(s, v, dir_ij, Wij, senders, receivers,
           Wi1, bi1, Wi2, bi2, Wm1, bm1, Wm2, bm2, Wvm):
    n_nodes = s.shape[0]
    s2d = s.reshape(n_nodes, H)
    v2d = v.reshape(n_nodes, 3 * H)
    wij2d = Wij.reshape(E, 3 * H)

    x = _stage1(s2d, Wi1, bi1, Wi2, bi2)

    rf = receivers.astype(jnp.float32)[:, None]
    sf = senders.astype(jnp.float32)[:, None]
    pk = jnp.concatenate(
        [rf, dir_ij, sf, jnp.zeros((E, 123), jnp.float32)], axis=1)
    spad = jnp.concatenate(
        [senders, jnp.full((EPAD - E,), 1 << 30, jnp.int32)])

    out = _sc_edge(x, v2d, wij2d, pk, spad)

    so, vo = _stage3(s2d, v2d, out, Wm1, bm1, Wm2, bm2, Wvm)
    return (so.reshape(n_nodes, 1, H), vo.reshape(n_nodes, 3, H))


# speedup vs baseline: 9.7797x; 3.5072x over previous
import functools

import jax
import jax.numpy as jnp
from jax import lax
from jax.experimental import pallas as pl
from jax.experimental.pallas import tpu as pltpu
from jax.experimental.pallas import tpu_sc as plsc

EPSILON = 1.0
EPS = 1e-8

N = 10000
E = 160000
H = 128

NC = 2
NS = 16
NW = NC * NS
NR = 2
NACC = 160
NPW = NR * NACC
NPADR = NW * NPW
SCH = 1024
NSCH = (E + SCH - 1) // SCH
EPAD = (NSCH + 1) * SCH
CB = 32
TRASH = SCH + CB
BKTW = SCH + CB + 16
F = 3 * H
OF = 4 * H


def _i16(val):
    return jnp.broadcast_to(val, (16,)).astype(jnp.int32)


def _sc_edge_body(x_hbm, v_hbm, wij_hbm, pk_hbm, snd_hbm, out_hbm,
                  sbuf, bkt, ibuf, rbuf, pkbuf, wbuf, xbuf, vbuf, acc,
                  sem_s, sem_pk, sem_w, sem_g):
    core = lax.axis_index("c")
    sub = lax.axis_index("s")
    w = core * NS + sub
    lane = lax.iota(jnp.int32, 16)
    z16 = jnp.zeros((16,), jnp.float32)

    for r in range(NR):
        lo = w * NPW + r * NACC

        def zrow(i, _):
            for g in range(OF // 16):
                acc[i, pl.ds(g * 16, 16)] = z16
            return 0

        lax.fori_loop(0, NACC + 1, zrow, 0)

        def make_chunkfn(cnt, lo):
            def chunkfn(ci, _):
                base = ci * CB
                for s2 in range(CB // 16):
                    ibuf[pl.ds(s2 * 16, 16)] = bkt[pl.ds(base + s2 * 16, 16)]
                cp_pk = pltpu.async_copy(pk_hbm.at[ibuf], pkbuf, sem_pk)
                cp_w = pltpu.async_copy(wij_hbm.at[ibuf], wbuf, sem_w)
                cp_pk.wait()
                for s2 in range(CB // 16):
                    rl = lane + s2 * 16
                    rv = plsc.load_gather(pkbuf, [rl, _i16(0)]).astype(jnp.int32)
                    rbuf[pl.ds(s2 * 16, 16)] = rv
                cp_x = pltpu.async_copy(x_hbm.at[rbuf], xbuf, sem_g)
                cp_v = pltpu.async_copy(v_hbm.at[rbuf], vbuf, sem_g)
                cp_w.wait()
                cp_x.wait()
                cp_v.wait()

                def edgefn(e, _):
                    validv = _i16(base + e) < _i16(cnt)
                    sndl = plsc.load_gather(
                        pkbuf, [_i16(e), _i16(4)]).astype(jnp.int32)
                    rowv = jnp.where(validv, sndl - _i16(lo), _i16(NACC))
                    d0 = plsc.load_gather(pkbuf, [_i16(e), _i16(1)])
                    d1 = plsc.load_gather(pkbuf, [_i16(e), _i16(2)])
                    d2 = plsc.load_gather(pkbuf, [_i16(e), _i16(3)])
                    m = []
                    for g in range(F // 16):
                        wv = wbuf[e, pl.ds(g * 16, 16)]
                        xv = xbuf[e, pl.ds(g * 16, 16)]
                        m.append(wv * xv)
                    for g in range(8):
                        plsc.addupdate_scatter(
                            acc, [rowv, _i16(g * 16) + lane], m[g])
                    for k, dk in enumerate((d0, d1, d2)):
                        for g in range(8):
                            vv = vbuf[e, pl.ds(k * 128 + g * 16, 16)]
                            val = m[8 + g] * dk + m[16 + g] * vv
                            plsc.addupdate_scatter(
                                acc, [rowv, _i16(128 + k * 128 + g * 16) + lane],
                                val)
                    return 0

                lax.fori_loop(0, CB, edgefn, 0)
                return 0
            return chunkfn

        pltpu.sync_copy(snd_hbm.at[pl.ds(0, SCH)], sbuf.at[0])

        def scanchunk(sc, carry, lo=lo):
            cp_n = pltpu.async_copy(
                snd_hbm.at[pl.ds((sc + 1) * SCH, SCH)],
                sbuf.at[(sc + 1) % 2], sem_s)

            def groupfn(g, cnt, sc=sc, lo=lo):
                sv = sbuf[sc % 2, pl.ds(g * 16, 16)]
                eid = _i16(sc * SCH + g * 16) + lane
                m = jnp.logical_and(sv >= _i16(lo), sv < _i16(lo + NACC))
                mi = m.astype(jnp.int32)
                offs = plsc.cumsum(mi)
                tgt = jnp.where(m, _i16(cnt) + offs - 1, _i16(TRASH))
                plsc.store_scatter(bkt, [tgt], eid)
                return cnt + jnp.sum(mi)

            total = lax.fori_loop(0, SCH // 16, groupfn, carry)

            nfull = total // CB
            lax.fori_loop(0, nfull, make_chunkfn(total, lo), 0)

            rem = total - nfull * CB
            for s2 in range(CB // 16):
                lv = bkt[pl.ds(nfull * CB + s2 * 16, 16)]
                bkt[pl.ds(s2 * 16, 16)] = lv
            cp_n.wait()
            return rem

        crem = lax.fori_loop(0, NSCH, scanchunk, 0)

        end = ((crem + CB - 1) // CB) * CB
        for rr in range(CB // 16):
            ii = _i16(crem + rr * 16) + lane
            tgt = jnp.where(ii < _i16(end), ii, _i16(TRASH))
            plsc.store_scatter(bkt, [tgt], jnp.zeros((16,), jnp.int32))
        lax.fori_loop(0, (crem + CB - 1) // CB, make_chunkfn(crem, lo), 0)

        pltpu.sync_copy(acc.at[pl.ds(0, NACC)],
                        out_hbm.at[pl.ds(lo, NACC)])


def _sc_edge(x, v2d, wij2d, pk, spad):
    mesh = plsc.VectorSubcoreMesh(core_axis_name="c", subcore_axis_name="s")
    fn = pl.kernel(
        _sc_edge_body,
        out_type=jax.ShapeDtypeStruct((NPADR, OF), jnp.float32),
        mesh=mesh,
        compiler_params=pltpu.CompilerParams(needs_layout_passes=False),
        scratch_types=[
            pltpu.VMEM((2, SCH), jnp.int32),
            pltpu.VMEM((BKTW,), jnp.int32),
            pltpu.VMEM((CB,), jnp.int32),
            pltpu.VMEM((CB,), jnp.int32),
            pltpu.VMEM((CB, 128), jnp.float32),
            pltpu.VMEM((CB, F), jnp.float32),
            pltpu.VMEM((CB, F), jnp.float32),
            pltpu.VMEM((CB, F), jnp.float32),
            pltpu.VMEM((NACC + 1, OF), jnp.float32),
            pltpu.SemaphoreType.DMA,
            pltpu.SemaphoreType.DMA,
            pltpu.SemaphoreType.DMA,
            pltpu.SemaphoreType.DMA,
        ],
    )
    return fn(x, v2d, wij2d, pk, spad)


def _stage1_body(s_ref, Wi1_ref, bi1_ref, Wi2_ref, bi2_ref, x_ref):
    s = s_ref[...]
    h = jnp.dot(s, Wi1_ref[...], preferred_element_type=jnp.float32) + bi1_ref[...]
    h = h * jax.nn.sigmoid(h)
    x_ref[...] = jnp.dot(h, Wi2_ref[...], preferred_element_type=jnp.float32) + bi2_ref[...]


def _stage1(s2d, Wi1, bi1, Wi2, bi2, block=1000):
    n = s2d.shape[0]
    grid = (n // block,)
    return pl.pallas_call(
        _stage1_body,
        grid=grid,
        in_specs=[
            pl.BlockSpec((block, H), lambda i: (i, 0)),
            pl.BlockSpec((H, H), lambda i: (0, 0)),
            pl.BlockSpec((H,), lambda i: (0,)),
            pl.BlockSpec((H, 3 * H), lambda i: (0, 0)),
            pl.BlockSpec((3 * H,), lambda i: (0,)),
        ],
        out_specs=pl.BlockSpec((block, 3 * H), lambda i: (i, 0)),
        out_shape=jax.ShapeDtypeStruct((n, 3 * H), jnp.float32),
    )(s2d, Wi1, bi1, Wi2, bi2)


def _stage3_body(s_ref, v_ref, o_ref, Wm1_ref, bm1_ref, Wm2_ref,
                 bm2_ref, Wvm_ref, so_ref, vo_ref, *, block):
    s0 = s_ref[...]
    v0 = v_ref[...].reshape(block, 3, H)
    o = o_ref[...]
    s = (s0 + o[:, :H]) * EPSILON
    v = (v0 + o[:, H:].reshape(block, 3, H)) * EPSILON
    vm = jnp.dot(v.reshape(block * 3, H), Wvm_ref[...],
                 preferred_element_type=jnp.float32).reshape(block, 3, 2 * H)
    v_l = vm[:, :, :H]
    v_r = vm[:, :, H:]
    v_norm = jnp.sqrt(jnp.sum(v_r * v_r, axis=1) + EPS)
    ts = jnp.concatenate([s, v_norm], axis=-1)
    hmid = jnp.dot(ts, Wm1_ref[...], preferred_element_type=jnp.float32) + bm1_ref[...]
    hmid = hmid * jax.nn.sigmoid(hmid)
    hh = jnp.dot(hmid, Wm2_ref[...], preferred_element_type=jnp.float32) + bm2_ref[...]
    ds2 = hh[:, :H]
    dv2u = hh[:, H:2 * H]
    dsv = hh[:, 2 * H:]
    dotrl = jnp.sum(v_r * v_l, axis=1)
    so_ref[...] = (s + ds2 + dsv * dotrl) * EPSILON
    vo_ref[...] = ((v + v_l * dv2u[:, None, :]) * EPSILON).reshape(block, 3 * H)


def _stage3(s2d, v2d, out, Wm1, bm1, Wm2, bm2, Wvm, block=1000):
    n = s2d.shape[0]
    grid = (n // block,)
    return pl.pallas_call(
        functools.partial(_stage3_body, block=block),
        grid=grid,
        in_specs=[
            pl.BlockSpec((block, H), lambda i: (i, 0)),
            pl.BlockSpec((block, 3 * H), lambda i: (i, 0)),
            pl.BlockSpec((block, OF), lambda i: (i, 0)),
            pl.BlockSpec((2 * H, H), lambda i: (0, 0)),
            pl.BlockSpec((H,), lambda i: (0,)),
            pl.BlockSpec((H, 3 * H), lambda i: (0, 0)),
            pl.BlockSpec((3 * H,), lambda i: (0,)),
            pl.BlockSpec((H, 2 * H), lambda i: (0, 0)),
        ],
        out_specs=[
            pl.BlockSpec((block, H), lambda i: (i, 0)),
            pl.BlockSpec((block, 3 * H), lambda i: (i, 0)),
        ],
        out_shape=[
            jax.ShapeDtypeStruct((n, H), jnp.float32),
            jax.ShapeDtypeStruct((n, 3 * H), jnp.float32),
        ],
    )(s2d, v2d, out, Wm1, bm1, Wm2, bm2, Wvm)


def kernel(s, v, dir_ij, Wij, senders, receivers,
           Wi1, bi1, Wi2, bi2, Wm1, bm1, Wm2, bm2, Wvm):
    n_nodes = s.shape[0]
    s2d = s.reshape(n_nodes, H)
    v2d = v.reshape(n_nodes, 3 * H)
    wij2d = Wij.reshape(E, 3 * H)

    x = _stage1(s2d, Wi1, bi1, Wi2, bi2)

    rf = receivers.astype(jnp.float32)[:, None]
    sf = senders.astype(jnp.float32)[:, None]
    pk = jnp.concatenate(
        [rf, dir_ij, sf, jnp.zeros((E, 123), jnp.float32)], axis=1)
    spad = jnp.concatenate(
        [senders, jnp.full((EPAD - E,), 1 << 30, jnp.int32)])

    out = _sc_edge(x, v2d, wij2d, pk, spad)

    so, vo = _stage3(s2d, v2d, out, Wm1, bm1, Wm2, bm2, Wvm)
    return (so.reshape(n_nodes, 1, H), vo.reshape(n_nodes, 3, H))
